# baseline (speedup 1.0000x reference)
import jax
import jax.numpy as jnp
from jax import lax
from jax.experimental import pallas as pl
from jax.experimental.pallas import tpu as pltpu
from jax.experimental.pallas import tpu_sc as plsc

_D = 512


def _body(l1_hbm, l0_hbm, out_hbm, buf):
    tid = lax.axis_index("s")

    @pl.when(tid < 8)
    def _():
        pltpu.sync_copy(l1_hbm.at[pl.ds(tid * 8, 8)], buf)
        pltpu.sync_copy(buf, out_hbm.at[pl.ds(tid * 8, 8)])

    @pl.when(tid >= 8)
    def _():
        m = tid - 8
        pltpu.sync_copy(l0_hbm.at[pl.ds(m * 8, 8), pl.ds(0, 1)], buf)
        pltpu.sync_copy(buf, out_hbm.at[pl.ds(64 + m * 8, 8)])


@jax.jit
def kernel(layer1, layer0):
    mesh = plsc.VectorSubcoreMesh(
        core_axis_name="c", subcore_axis_name="s", num_cores=1
    )
    f = pl.kernel(
        _body,
        out_type=jax.ShapeDtypeStruct((128, 1, 2, _D), jnp.float32),
        mesh=mesh,
        scratch_types=[pltpu.VMEM((8, 1, 2, _D), jnp.float32)],
    )
    l1_p = layer1.reshape(8192, 1, 2, _D)
    l0_q = layer0.reshape(4096, 2, 2, _D)
    return f(l1_p, l0_q).reshape(256, _D)


# speedup vs baseline: 5.4159x; 5.4159x over previous
import jax
import jax.numpy as jnp
from jax.experimental import pallas as pl
from jax.experimental.pallas import tpu as pltpu
from jax.experimental.pallas import tpu_sc as plsc

_D = 512


def _body(l1_hbm, l0_hbm, out_hbm, buf, sem):
    copies = [
        pltpu.make_async_copy(
            l1_hbm.at[pl.ds(0, 128)], buf.at[pl.ds(0, 128)], sem
        )
    ]
    for k in range(64):
        copies.append(
            pltpu.make_async_copy(
                l0_hbm.at[pl.ds(4 * k, 2)], buf.at[pl.ds(128 + 2 * k, 2)], sem
            )
        )
    for c in copies:
        c.start()
    for c in copies:
        c.wait()
    pltpu.sync_copy(buf, out_hbm)


@jax.jit
def kernel(layer1, layer0):
    mesh = plsc.ScalarSubcoreMesh(axis_name="c", num_cores=1)
    f = pl.kernel(
        _body,
        out_type=jax.ShapeDtypeStruct((256, _D), jnp.float32),
        mesh=mesh,
        scratch_types=[
            pltpu.VMEM_SHARED((256, _D), jnp.float32),
            pltpu.SemaphoreType.DMA,
        ],
    )
    return f(layer1, layer0)


# speedup vs baseline: 5.5139x; 1.0181x over previous
import jax
import jax.numpy as jnp
from jax.experimental import pallas as pl
from jax.experimental.pallas import tpu as pltpu
from jax.experimental.pallas import tpu_sc as plsc

_D = 512


def _body(l1_hbm, l0_hbm, out_hbm, buf, sem1, sem0, semo):
    c1 = pltpu.make_async_copy(
        l1_hbm.at[pl.ds(0, 128)], buf.at[pl.ds(0, 128)], sem1
    )
    c1.start()
    pairs = [
        pltpu.make_async_copy(
            l0_hbm.at[pl.ds(4 * k, 2)], buf.at[pl.ds(128 + 2 * k, 2)], sem0
        )
        for k in range(64)
    ]
    for c in pairs:
        c.start()
    c1.wait()
    o1 = pltpu.make_async_copy(
        buf.at[pl.ds(0, 128)], out_hbm.at[pl.ds(0, 128)], semo
    )
    o1.start()
    for c in pairs:
        c.wait()
    o0 = pltpu.make_async_copy(
        buf.at[pl.ds(128, 128)], out_hbm.at[pl.ds(128, 128)], semo
    )
    o0.start()
    o1.wait()
    o0.wait()


@jax.jit
def kernel(layer1, layer0):
    mesh = plsc.ScalarSubcoreMesh(axis_name="c", num_cores=1)
    f = pl.kernel(
        _body,
        out_type=jax.ShapeDtypeStruct((256, _D), jnp.float32),
        mesh=mesh,
        scratch_types=[
            pltpu.VMEM_SHARED((256, _D), jnp.float32),
            pltpu.SemaphoreType.DMA,
            pltpu.SemaphoreType.DMA,
            pltpu.SemaphoreType.DMA,
        ],
    )
    return f(layer1, layer0)
